# baseline (speedup 1.0000x reference)
import functools

import jax
import jax.numpy as jnp
import numpy as np
from jax.experimental import pallas as pl
from jax.experimental.pallas import tpu as pltpu

_V_TILE = 4096
_B_CHUNKS = 2
_TINY = float(np.finfo(np.float32).tiny)

_K1 = 42
_K2 = 0 ^ 42 ^ 0x1BD11BDA
_KS = (0, _K1, _K2)
_ROT_A = (13, 15, 26, 6)
_ROT_B = (17, 29, 16, 24)


def _rotl(x, r):
    return (x << jnp.uint32(r)) | (x >> jnp.uint32(32 - r))


def _threefry_bits(flat_u32):
    x1 = flat_u32 + jnp.uint32(_KS[1])
    x0 = x1
    x1 = _rotl(x1, _ROT_A[0]) ^ x0
    rots = (_ROT_A, _ROT_B)
    for i in range(5):
        for r in (rots[i % 2][1:] if i == 0 else rots[i % 2]):
            x0 = x0 + x1
            x1 = _rotl(x1, r)
            x1 = x1 ^ x0
        x0 = x0 + jnp.uint32(_KS[(i + 1) % 3])
        x1 = x1 + jnp.uint32((_KS[(i + 2) % 3] + i + 1) & 0xFFFFFFFF)
    return x0 ^ x1


def _gumbel_from_bits(bits):
    fb = (bits >> jnp.uint32(9)) | jnp.uint32(0x3F800000)
    f = jax.lax.bitcast_convert_type(fb, jnp.float32) - jnp.float32(1.0)
    u = f + jnp.float32(_TINY)
    return -jnp.log(-jnp.log(u))


def _fused_kernel(n_total, n_tiles,
                  x_ref, w1_ref, b1_ref, w2_ref, b2_ref,
                  sample_ref, logp_ref,
                  h_scr, flat_scr, m_scr, s_scr, bestv_scr, bidx_scr,
                  blog_scr):
    c = pl.program_id(0)
    t = pl.program_id(1)
    b = x_ref.shape[0]
    v = _V_TILE
    neg_inf = jnp.float32(-jnp.inf)

    @pl.when(t == 0)
    def _init():
        h = jax.lax.dot_general(
            x_ref[...], w1_ref[...], (((1,), (1,)), ((), ())),
            preferred_element_type=jnp.float32)
        h_scr[...] = jnp.maximum(h + b1_ref[...], 0.0)
        row = jax.lax.broadcasted_iota(jnp.int32, (b, v), 0) + c * b
        lane = jax.lax.broadcasted_iota(jnp.int32, (b, v), 1)
        flat_scr[...] = (row * n_total + lane).astype(jnp.uint32)
        m_scr[...] = jnp.full((b, 1), neg_inf, jnp.float32)
        s_scr[...] = jnp.zeros((b, 1), jnp.float32)
        bestv_scr[...] = jnp.full((b, 1), neg_inf, jnp.float32)
        bidx_scr[...] = jnp.zeros((b, 1), jnp.int32)
        blog_scr[...] = jnp.zeros((b, 1), jnp.float32)

    logits = jax.lax.dot_general(
        h_scr[...], w2_ref[...], (((1,), (1,)), ((), ())),
        preferred_element_type=jnp.float32) + b2_ref[...]

    col = jax.lax.broadcasted_iota(jnp.int32, (b, v), 1) + t * v
    valid = col < n_total
    logits = jnp.where(valid, logits, neg_inf)

    g = _gumbel_from_bits(_threefry_bits(flat_scr[...] + jnp.uint32(t * v)))
    pert = g + logits

    pmax = jnp.max(pert, axis=1, keepdims=True)
    pidx_local = jnp.argmax(pert, axis=1).astype(jnp.int32)[:, None]
    pidx = pidx_local + t * v
    lane = jax.lax.broadcasted_iota(jnp.int32, (b, v), 1)
    logit_at = jnp.sum(jnp.where(lane == pidx_local, logits, 0.0),
                       axis=1, keepdims=True)

    m_old = m_scr[...]
    m_new = jnp.maximum(m_old, pmax + jnp.float32(4.5))
    tsum = jnp.sum(jnp.exp(logits - m_new), axis=1, keepdims=True)
    s_scr[...] = s_scr[...] * jnp.exp(m_old - m_new) + tsum
    m_scr[...] = m_new

    upd = pmax > bestv_scr[...]
    bestv_scr[...] = jnp.where(upd, pmax, bestv_scr[...])
    bidx_scr[...] = jnp.where(upd, pidx, bidx_scr[...])
    blog_scr[...] = jnp.where(upd, logit_at, blog_scr[...])

    @pl.when(t == n_tiles - 1)
    def _finish():
        sample_ref[...] = bidx_scr[...]
        logp_ref[...] = (blog_scr[...] - m_scr[...]) - jnp.log(s_scr[...])


def kernel(x, W1, b1, W2, b2, batch_size=1):
    bsz, e = x.shape
    h_dim = W1.shape[0]
    n = W2.shape[0]
    n_tiles = (n + _V_TILE - 1) // _V_TILE
    bc = bsz // _B_CHUNKS

    b1r = b1.reshape(1, h_dim)
    b2r = b2.reshape(1, n)

    sample2d, logp2d = pl.pallas_call(
        functools.partial(_fused_kernel, n, n_tiles),
        grid=(_B_CHUNKS, n_tiles),
        in_specs=[
            pl.BlockSpec((bc, e), lambda c, t: (c, 0)),
            pl.BlockSpec((h_dim, e), lambda c, t: (0, 0)),
            pl.BlockSpec((1, h_dim), lambda c, t: (0, 0)),
            pl.BlockSpec((_V_TILE, h_dim), lambda c, t: (t, 0)),
            pl.BlockSpec((1, _V_TILE), lambda c, t: (0, t)),
        ],
        out_specs=[
            pl.BlockSpec((bc, 1), lambda c, t: (c, 0)),
            pl.BlockSpec((bc, 1), lambda c, t: (c, 0)),
        ],
        out_shape=[
            jax.ShapeDtypeStruct((bsz, 1), jnp.int32),
            jax.ShapeDtypeStruct((bsz, 1), jnp.float32),
        ],
        scratch_shapes=[
            pltpu.VMEM((bc, h_dim), jnp.float32),
            pltpu.VMEM((bc, _V_TILE), jnp.uint32),
            pltpu.VMEM((bc, 1), jnp.float32),
            pltpu.VMEM((bc, 1), jnp.float32),
            pltpu.VMEM((bc, 1), jnp.float32),
            pltpu.VMEM((bc, 1), jnp.int32),
            pltpu.VMEM((bc, 1), jnp.float32),
        ],
        compiler_params=pltpu.CompilerParams(
            dimension_semantics=("parallel", "arbitrary"),
        ),
    )(x, W1, b1r, W2, b2r)

    return (sample2d.reshape(bsz), logp2d.reshape(bsz))


# speedup vs baseline: 1.2974x; 1.2974x over previous
import functools

import jax
import jax.numpy as jnp
import numpy as np
from jax.experimental import pallas as pl
from jax.experimental.pallas import tpu as pltpu

_V_TILE = 2048
_B_CHUNKS = 2
_TINY = float(np.finfo(np.float32).tiny)

_K1 = 42
_K2 = 0 ^ 42 ^ 0x1BD11BDA
_KS = (0, _K1, _K2)
_ROT_A = (13, 15, 26, 6)
_ROT_B = (17, 29, 16, 24)


def _rotl(x, r):
    return (x << jnp.uint32(r)) | (x >> jnp.uint32(32 - r))


def _threefry_bits(flat_u32):
    x1 = flat_u32 + jnp.uint32(_KS[1])
    x0 = x1
    x1 = _rotl(x1, _ROT_A[0]) ^ x0
    rots = (_ROT_A, _ROT_B)
    for i in range(5):
        for r in (rots[i % 2][1:] if i == 0 else rots[i % 2]):
            x0 = x0 + x1
            x1 = _rotl(x1, r)
            x1 = x1 ^ x0
        x0 = x0 + jnp.uint32(_KS[(i + 1) % 3])
        x1 = x1 + jnp.uint32((_KS[(i + 2) % 3] + i + 1) & 0xFFFFFFFF)
    return x0 ^ x1


def _gumbel_from_bits(bits):
    fb = (bits >> jnp.uint32(9)) | jnp.uint32(0x3F800000)
    f = jax.lax.bitcast_convert_type(fb, jnp.float32) - jnp.float32(1.0)
    u = f + jnp.float32(_TINY)
    return -jnp.log(-jnp.log(u))


def _fused_kernel(n_total, n_tiles,
                  x_ref, w1_ref, b1_ref, w2_ref, b2_ref,
                  sample_ref, logp_ref,
                  h_scr, flat_scr, m_scr, s_scr, bestv_scr, bidx_scr,
                  blog_scr):
    c = pl.program_id(0)
    t = pl.program_id(1)
    b = x_ref.shape[0]
    v = _V_TILE
    neg_inf = jnp.float32(-jnp.inf)

    @pl.when(t == 0)
    def _init():
        h = jax.lax.dot_general(
            x_ref[...], w1_ref[...], (((1,), (1,)), ((), ())),
            preferred_element_type=jnp.float32)
        h_scr[...] = jnp.maximum(h + b1_ref[...], 0.0)
        row = jax.lax.broadcasted_iota(jnp.int32, (b, v), 0) + c * b
        lane = jax.lax.broadcasted_iota(jnp.int32, (b, v), 1)
        flat_scr[...] = (row * n_total + lane).astype(jnp.uint32)
        m_scr[...] = jnp.full((b, 1), neg_inf, jnp.float32)
        s_scr[...] = jnp.zeros((b, 1), jnp.float32)
        bestv_scr[...] = jnp.full((b, 1), neg_inf, jnp.float32)
        bidx_scr[...] = jnp.zeros((b, 1), jnp.int32)
        blog_scr[...] = jnp.zeros((b, 1), jnp.float32)

    logits = jax.lax.dot_general(
        h_scr[...], w2_ref[...], (((1,), (1,)), ((), ())),
        preferred_element_type=jnp.float32) + b2_ref[...]

    col = jax.lax.broadcasted_iota(jnp.int32, (b, v), 1) + t * v
    valid = col < n_total
    logits = jnp.where(valid, logits, neg_inf)

    g = _gumbel_from_bits(_threefry_bits(flat_scr[...] + jnp.uint32(t * v)))
    pert = g + logits

    pmax = jnp.max(pert, axis=1, keepdims=True)
    pidx_local = jnp.argmax(pert, axis=1).astype(jnp.int32)[:, None]
    pidx = pidx_local + t * v
    lane = jax.lax.broadcasted_iota(jnp.int32, (b, v), 1)
    logit_at = jnp.sum(jnp.where(lane == pidx_local, logits, 0.0),
                       axis=1, keepdims=True)

    m_old = m_scr[...]
    m_new = jnp.maximum(m_old, pmax + jnp.float32(4.5))
    tsum = jnp.sum(jnp.exp(logits - m_new), axis=1, keepdims=True)
    s_scr[...] = s_scr[...] * jnp.exp(m_old - m_new) + tsum
    m_scr[...] = m_new

    upd = pmax > bestv_scr[...]
    bestv_scr[...] = jnp.where(upd, pmax, bestv_scr[...])
    bidx_scr[...] = jnp.where(upd, pidx, bidx_scr[...])
    blog_scr[...] = jnp.where(upd, logit_at, blog_scr[...])

    @pl.when(t == n_tiles - 1)
    def _finish():
        sample_ref[...] = bidx_scr[...]
        logp_ref[...] = (blog_scr[...] - m_scr[...]) - jnp.log(s_scr[...])


def kernel(x, W1, b1, W2, b2, batch_size=1):
    bsz, e = x.shape
    h_dim = W1.shape[0]
    n = W2.shape[0]
    n_tiles = (n + _V_TILE - 1) // _V_TILE
    bc = bsz // _B_CHUNKS

    b1r = b1.reshape(1, h_dim)
    b2r = b2.reshape(1, n)

    sample2d, logp2d = pl.pallas_call(
        functools.partial(_fused_kernel, n, n_tiles),
        grid=(_B_CHUNKS, n_tiles),
        in_specs=[
            pl.BlockSpec((bc, e), lambda c, t: (c, 0)),
            pl.BlockSpec((h_dim, e), lambda c, t: (0, 0)),
            pl.BlockSpec((1, h_dim), lambda c, t: (0, 0)),
            pl.BlockSpec((_V_TILE, h_dim), lambda c, t: (t, 0)),
            pl.BlockSpec((1, _V_TILE), lambda c, t: (0, t)),
        ],
        out_specs=[
            pl.BlockSpec((bc, 1), lambda c, t: (c, 0)),
            pl.BlockSpec((bc, 1), lambda c, t: (c, 0)),
        ],
        out_shape=[
            jax.ShapeDtypeStruct((bsz, 1), jnp.int32),
            jax.ShapeDtypeStruct((bsz, 1), jnp.float32),
        ],
        scratch_shapes=[
            pltpu.VMEM((bc, h_dim), jnp.float32),
            pltpu.VMEM((bc, _V_TILE), jnp.uint32),
            pltpu.VMEM((bc, 1), jnp.float32),
            pltpu.VMEM((bc, 1), jnp.float32),
            pltpu.VMEM((bc, 1), jnp.float32),
            pltpu.VMEM((bc, 1), jnp.int32),
            pltpu.VMEM((bc, 1), jnp.float32),
        ],
        compiler_params=pltpu.CompilerParams(
            dimension_semantics=("parallel", "arbitrary"),
        ),
    )(x, W1, b1r, W2, b2r)

    return (sample2d.reshape(bsz), logp2d.reshape(bsz))
